# baseline (speedup 1.0000x reference)
import functools

import jax
import jax.numpy as jnp
from jax import lax
from jax.experimental import pallas as pl
from jax.experimental.pallas import tpu as pltpu
from jax.experimental.pallas import tpu_sc as plsc

_INFO = plsc.get_sparse_core_info()
_NC = _INFO.num_cores
_NS = _INFO.num_subcores
_NW = _NC * _NS
_LANES = _INFO.num_lanes


def kernel(user_id, item_id, hist_item, price, user_table, item_table,
           hist_table):
    B = user_id.shape[0]
    L = hist_item.shape[1]
    D = user_table.shape[1]
    OUTW = 3 * D + 1
    RPW = B // _NW
    CB = 16
    NCH = RPW // CB
    NH = D // _LANES

    mesh = plsc.VectorSubcoreMesh(core_axis_name="c", subcore_axis_name="s")

    @functools.partial(
        pl.kernel,
        out_type=jax.ShapeDtypeStruct((B, OUTW), jnp.float32),
        mesh=mesh,
        compiler_params=pltpu.CompilerParams(
            needs_layout_passes=False, use_tc_tiling_on_sc=False),
        scratch_types=[
            pltpu.VMEM((RPW,), jnp.int32),
            pltpu.VMEM((RPW,), jnp.int32),
            pltpu.VMEM((RPW, L), jnp.int32),
            pltpu.VMEM((RPW,), jnp.float32),
            pltpu.VMEM((CB, D), jnp.float32),
            pltpu.VMEM((CB, D), jnp.float32),
            pltpu.VMEM((CB, D), jnp.float32),
            pltpu.VMEM((CB, D), jnp.float32),
            pltpu.VMEM((CB, L, D), jnp.float32),
            pltpu.VMEM((CB, L, D), jnp.float32),
            pltpu.VMEM((CB, OUTW), jnp.float32),
            pltpu.VMEM((CB, OUTW), jnp.float32),
            pltpu.SemaphoreType.DMA,
            pltpu.SemaphoreType.DMA,
            pltpu.SemaphoreType.DMA,
            pltpu.SemaphoreType.DMA,
            pltpu.SemaphoreType.DMA,
            pltpu.SemaphoreType.DMA,
        ],
    )
    def _emb(uid, iid, hid, pr, ut, it, ht, out,
             uidx, iidx, hidx, pst, su0, su1, si0, si1, hb0, hb1,
             st0, st1, semh0, semh1, semg0, semg1, semo0, semo1):
        su = (su0, su1)
        si = (si0, si1)
        hb = (hb0, hb1)
        st = (st0, st1)
        semh = (semh0, semh1)
        semg = (semg0, semg1)
        semo = (semo0, semo1)

        wid = lax.axis_index("s") * _NC + lax.axis_index("c")
        base = wid * RPW

        cps = [
            pltpu.async_copy(uid.at[pl.ds(base, RPW)], uidx, semg0),
            pltpu.async_copy(iid.at[pl.ds(base, RPW)], iidx, semg0),
            pltpu.async_copy(hid.at[pl.ds(base, RPW)], hidx, semg0),
            pltpu.async_copy(pr.at[pl.ds(base, RPW)], pst, semg0),
        ]
        for c in cps:
            c.wait()

        iota16 = lax.broadcasted_iota(jnp.int32, (_LANES,), 0)
        col_last = jnp.full((_LANES,), OUTW - 1, jnp.int32)

        def fire(g, p):
            r0 = g * CB
            for c in range(CB):
                pltpu.async_copy(ht.at[hidx.at[r0 + c]], hb[p].at[c], semh[p])
            pltpu.async_copy(ut.at[uidx.at[pl.ds(r0, CB)]], su[p], semg[p])
            pltpu.async_copy(it.at[iidx.at[pl.ds(r0, CB)]], si[p], semg[p])

        def drain(g, p):
            r0 = g * CB
            for c in range(CB):
                pltpu.make_async_copy(
                    ht.at[hidx.at[r0 + c]], hb[p].at[c], semh[p]).wait()
            pltpu.make_async_copy(
                ut.at[uidx.at[pl.ds(r0, CB)]], su[p], semg[p]).wait()
            pltpu.make_async_copy(
                it.at[iidx.at[pl.ds(r0, CB)]], si[p], semg[p]).wait()

        def compute(g, p):
            r0 = g * CB

            @pl.when(g >= 2)
            def _():
                pltpu.make_async_copy(
                    st[p], out.at[pl.ds(base + (g - 2) * CB, CB)],
                    semo[p]).wait()

            for c in range(CB):
                for h in range(NH):
                    o = h * _LANES
                    acc = [hb[p][c, l, pl.ds(o, _LANES)] for l in range(4)]
                    for l in range(4, L):
                        acc[l % 4] = acc[l % 4] + hb[p][c, l, pl.ds(o, _LANES)]
                    a = (acc[0] + acc[1]) + (acc[2] + acc[3])
                    st[p][c, pl.ds(2 * D + o, _LANES)] = a
                    st[p][c, pl.ds(o, _LANES)] = su[p][c, pl.ds(o, _LANES)]
                    st[p][c, pl.ds(D + o, _LANES)] = si[p][c, pl.ds(o, _LANES)]

            plsc.store_scatter(st[p], [iota16, col_last], pst[pl.ds(r0, CB)])
            pltpu.async_copy(st[p], out.at[pl.ds(base + r0, CB)], semo[p])

        NP = NCH // 2
        fire(0, 0)

        def pair(gp, _):
            g0 = gp * 2
            fire(g0 + 1, 1)
            drain(g0, 0)
            compute(g0, 0)

            @pl.when(gp < NP - 1)
            def _():
                fire(g0 + 2, 0)

            drain(g0 + 1, 1)
            compute(g0 + 1, 1)
            return 0

        lax.fori_loop(0, NP, pair, 0)

        pltpu.make_async_copy(
            st0, out.at[pl.ds(base + (NCH - 2) * CB, CB)], semo0).wait()
        pltpu.make_async_copy(
            st1, out.at[pl.ds(base + (NCH - 1) * CB, CB)], semo1).wait()

    return _emb(user_id, item_id, hist_item, price, user_table, item_table,
                hist_table)


# speedup vs baseline: 1.0453x; 1.0453x over previous
import functools

import jax
import jax.numpy as jnp
from jax import lax
from jax.experimental import pallas as pl
from jax.experimental.pallas import tpu as pltpu
from jax.experimental.pallas import tpu_sc as plsc

_INFO = plsc.get_sparse_core_info()
_NC = _INFO.num_cores
_NS = _INFO.num_subcores
_NW = _NC * _NS
_LANES = _INFO.num_lanes


def kernel(user_id, item_id, hist_item, price, user_table, item_table,
           hist_table):
    B = user_id.shape[0]
    L = hist_item.shape[1]
    D = user_table.shape[1]
    OUTW = 3 * D + 1
    RPW = B // _NW
    CB = 16
    NCH = RPW // CB
    NH = D // _LANES
    HPC = CB * L
    GS = [(k * 128, min(128, HPC - k * 128)) for k in range((HPC + 127) // 128)]

    hist_flat = hist_item.reshape(-1)

    mesh = plsc.VectorSubcoreMesh(core_axis_name="c", subcore_axis_name="s")

    @functools.partial(
        pl.kernel,
        out_type=jax.ShapeDtypeStruct((B, OUTW), jnp.float32),
        mesh=mesh,
        compiler_params=pltpu.CompilerParams(
            needs_layout_passes=False, use_tc_tiling_on_sc=False),
        scratch_types=[
            pltpu.VMEM((RPW,), jnp.int32),
            pltpu.VMEM((RPW,), jnp.int32),
            pltpu.VMEM((RPW * L,), jnp.int32),
            pltpu.VMEM((RPW,), jnp.float32),
            pltpu.VMEM((CB, D), jnp.float32),
            pltpu.VMEM((CB, D), jnp.float32),
            pltpu.VMEM((CB, D), jnp.float32),
            pltpu.VMEM((CB, D), jnp.float32),
            pltpu.VMEM((HPC, D), jnp.float32),
            pltpu.VMEM((HPC, D), jnp.float32),
            pltpu.VMEM((CB, OUTW), jnp.float32),
            pltpu.VMEM((CB, OUTW), jnp.float32),
            pltpu.SemaphoreType.DMA,
            pltpu.SemaphoreType.DMA,
            pltpu.SemaphoreType.DMA,
            pltpu.SemaphoreType.DMA,
            pltpu.SemaphoreType.DMA,
            pltpu.SemaphoreType.DMA,
        ],
    )
    def _emb(uid, iid, hid, pr, ut, it, ht, out,
             uidx, iidx, hidx, pst, su0, su1, si0, si1, hb0, hb1,
             st0, st1, semh0, semh1, semg0, semg1, semo0, semo1):
        su = (su0, su1)
        si = (si0, si1)
        hb = (hb0, hb1)
        st = (st0, st1)
        semh = (semh0, semh1)
        semg = (semg0, semg1)
        semo = (semo0, semo1)

        wid = lax.axis_index("s") * _NC + lax.axis_index("c")
        base = wid * RPW

        cps = [
            pltpu.async_copy(uid.at[pl.ds(base, RPW)], uidx, semg0),
            pltpu.async_copy(iid.at[pl.ds(base, RPW)], iidx, semg0),
            pltpu.async_copy(hid.at[pl.ds(base * L, RPW * L)], hidx, semg0),
            pltpu.async_copy(pr.at[pl.ds(base, RPW)], pst, semg0),
        ]
        for c in cps:
            c.wait()

        iota16 = lax.broadcasted_iota(jnp.int32, (_LANES,), 0)
        col_last = jnp.full((_LANES,), OUTW - 1, jnp.int32)

        def fire(g, p):
            r0 = g * CB
            h0 = r0 * L
            for (o, n) in GS:
                pltpu.async_copy(ht.at[hidx.at[pl.ds(h0 + o, n)]],
                                 hb[p].at[pl.ds(o, n)], semh[p])
            pltpu.async_copy(ut.at[uidx.at[pl.ds(r0, CB)]], su[p], semg[p])
            pltpu.async_copy(it.at[iidx.at[pl.ds(r0, CB)]], si[p], semg[p])

        def drain(g, p):
            r0 = g * CB
            h0 = r0 * L
            for (o, n) in GS:
                pltpu.make_async_copy(ht.at[hidx.at[pl.ds(h0 + o, n)]],
                                      hb[p].at[pl.ds(o, n)], semh[p]).wait()
            pltpu.make_async_copy(
                ut.at[uidx.at[pl.ds(r0, CB)]], su[p], semg[p]).wait()
            pltpu.make_async_copy(
                it.at[iidx.at[pl.ds(r0, CB)]], si[p], semg[p]).wait()

        def compute(g, p):
            r0 = g * CB

            @pl.when(g >= 2)
            def _():
                pltpu.make_async_copy(
                    st[p], out.at[pl.ds(base + (g - 2) * CB, CB)],
                    semo[p]).wait()

            for c in range(CB):
                for h in range(NH):
                    o = h * _LANES
                    acc = [hb[p][c * L + l, pl.ds(o, _LANES)]
                           for l in range(4)]
                    for l in range(4, L):
                        acc[l % 4] = (acc[l % 4]
                                      + hb[p][c * L + l, pl.ds(o, _LANES)])
                    a = (acc[0] + acc[1]) + (acc[2] + acc[3])
                    st[p][c, pl.ds(2 * D + o, _LANES)] = a
                    st[p][c, pl.ds(o, _LANES)] = su[p][c, pl.ds(o, _LANES)]
                    st[p][c, pl.ds(D + o, _LANES)] = si[p][c, pl.ds(o, _LANES)]

            plsc.store_scatter(st[p], [iota16, col_last], pst[pl.ds(r0, CB)])
            pltpu.async_copy(st[p], out.at[pl.ds(base + r0, CB)], semo[p])

        NP = NCH // 2
        fire(0, 0)

        def pair(gp, _):
            g0 = gp * 2
            fire(g0 + 1, 1)
            drain(g0, 0)
            compute(g0, 0)

            @pl.when(gp < NP - 1)
            def _():
                fire(g0 + 2, 0)

            drain(g0 + 1, 1)
            compute(g0 + 1, 1)
            return 0

        lax.fori_loop(0, NP, pair, 0)

        pltpu.make_async_copy(
            st0, out.at[pl.ds(base + (NCH - 2) * CB, CB)], semo0).wait()
        pltpu.make_async_copy(
            st1, out.at[pl.ds(base + (NCH - 1) * CB, CB)], semo1).wait()

    return _emb(user_id, item_id, hist_flat, price, user_table, item_table,
                hist_table)


# speedup vs baseline: 1.5997x; 1.5304x over previous
import functools

import jax
import jax.numpy as jnp
from jax import lax
from jax.experimental import pallas as pl
from jax.experimental.pallas import tpu as pltpu
from jax.experimental.pallas import tpu_sc as plsc

_INFO = plsc.get_sparse_core_info()
_NC = _INFO.num_cores
_NS = _INFO.num_subcores
_NW = _NC * _NS
_LANES = _INFO.num_lanes


def kernel(user_id, item_id, hist_item, price, user_table, item_table,
           hist_table):
    B = user_id.shape[0]
    L = hist_item.shape[1]
    D = user_table.shape[1]
    OUTW = 3 * D + 1
    RPW = B // _NW
    CB = 16
    NCH = RPW // CB
    NH = D // _LANES
    HPC = CB * L
    GS = [(k * 128, min(128, HPC - k * 128)) for k in range((HPC + 127) // 128)]

    hist_flat = hist_item.reshape(-1)

    mesh = plsc.VectorSubcoreMesh(core_axis_name="c", subcore_axis_name="s")

    @functools.partial(
        pl.kernel,
        out_type=jax.ShapeDtypeStruct((B, OUTW), jnp.float32),
        mesh=mesh,
        compiler_params=pltpu.CompilerParams(
            needs_layout_passes=False, use_tc_tiling_on_sc=False),
        scratch_types=[
            pltpu.VMEM((RPW,), jnp.int32),
            pltpu.VMEM((RPW,), jnp.int32),
            pltpu.VMEM((RPW * L,), jnp.int32),
            pltpu.VMEM((RPW,), jnp.float32),
            pltpu.VMEM((CB, D), jnp.float32),
            pltpu.VMEM((CB, D), jnp.float32),
            pltpu.VMEM((CB, D), jnp.float32),
            pltpu.VMEM((CB, D), jnp.float32),
            pltpu.VMEM((HPC, D), jnp.float32),
            pltpu.VMEM((HPC, D), jnp.float32),
            pltpu.VMEM((CB, OUTW), jnp.float32),
            pltpu.VMEM((CB, OUTW), jnp.float32),
            pltpu.SemaphoreType.DMA,
            pltpu.SemaphoreType.DMA,
            pltpu.SemaphoreType.DMA,
            pltpu.SemaphoreType.DMA,
            pltpu.SemaphoreType.DMA,
            pltpu.SemaphoreType.DMA,
        ],
    )
    def _emb(uid, iid, hid, pr, ut, it, ht, out,
             uidx, iidx, hidx, pst, su0, su1, si0, si1, hb0, hb1,
             st0, st1, semh0, semh1, semg0, semg1, semo0, semo1):
        su = (su0, su1)
        si = (si0, si1)
        hb = (hb0, hb1)
        st = (st0, st1)
        semh = (semh0, semh1)
        semg = (semg0, semg1)
        semo = (semo0, semo1)

        wid = lax.axis_index("s") * _NC + lax.axis_index("c")
        base = wid * RPW

        cps = [
            pltpu.async_copy(uid.at[pl.ds(base, RPW)], uidx, semg0),
            pltpu.async_copy(iid.at[pl.ds(base, RPW)], iidx, semg0),
            pltpu.async_copy(hid.at[pl.ds(base * L, RPW * L)], hidx, semg0),
            pltpu.async_copy(pr.at[pl.ds(base, RPW)], pst, semg0),
        ]
        for c in cps:
            c.wait()

        iota16 = lax.broadcasted_iota(jnp.int32, (_LANES,), 0)
        col_last = jnp.full((_LANES,), OUTW - 1, jnp.int32)

        def fire(g, p):
            r0 = g * CB
            h0 = r0 * L
            for (o, n) in GS:
                pltpu.async_copy(ht.at[hidx.at[pl.ds(h0 + o, n)]],
                                 hb[p].at[pl.ds(o, n)], semh[p])
            pltpu.async_copy(ut.at[uidx.at[pl.ds(r0, CB)]], su[p], semg[p])
            pltpu.async_copy(it.at[iidx.at[pl.ds(r0, CB)]], si[p], semg[p])

        def drain(g, p):
            r0 = g * CB
            h0 = r0 * L
            for (o, n) in GS:
                pltpu.make_async_copy(ht.at[hidx.at[pl.ds(h0 + o, n)]],
                                      hb[p].at[pl.ds(o, n)], semh[p]).wait()
            pltpu.make_async_copy(
                ut.at[uidx.at[pl.ds(r0, CB)]], su[p], semg[p]).wait()
            pltpu.make_async_copy(
                it.at[iidx.at[pl.ds(r0, CB)]], si[p], semg[p]).wait()

        def compute(g, p):
            r0 = g * CB

            @pl.when(g >= 2)
            def _():
                pltpu.make_async_copy(
                    st[p], out.at[pl.ds(base + (g - 2) * CB, CB)],
                    semo[p]).wait()

            def crow(c, _):
                for h in range(NH):
                    o = h * _LANES
                    acc = [hb[p][c * L + l, pl.ds(o, _LANES)]
                           for l in range(4)]
                    for l in range(4, L):
                        acc[l % 4] = (acc[l % 4]
                                      + hb[p][c * L + l, pl.ds(o, _LANES)])
                    a = (acc[0] + acc[1]) + (acc[2] + acc[3])
                    st[p][c, pl.ds(2 * D + o, _LANES)] = a
                    st[p][c, pl.ds(o, _LANES)] = su[p][c, pl.ds(o, _LANES)]
                    st[p][c, pl.ds(D + o, _LANES)] = si[p][c, pl.ds(o, _LANES)]
                return 0

            lax.fori_loop(0, CB, crow, 0)

            plsc.store_scatter(st[p], [iota16, col_last], pst[pl.ds(r0, CB)])
            pltpu.async_copy(st[p], out.at[pl.ds(base + r0, CB)], semo[p])

        NP = NCH // 2
        fire(0, 0)

        def pair(gp, _):
            g0 = gp * 2
            fire(g0 + 1, 1)
            drain(g0, 0)
            compute(g0, 0)

            @pl.when(gp < NP - 1)
            def _():
                fire(g0 + 2, 0)

            drain(g0 + 1, 1)
            compute(g0 + 1, 1)
            return 0

        lax.fori_loop(0, NP, pair, 0)

        pltpu.make_async_copy(
            st0, out.at[pl.ds(base + (NCH - 2) * CB, CB)], semo0).wait()
        pltpu.make_async_copy(
            st1, out.at[pl.ds(base + (NCH - 1) * CB, CB)], semo1).wait()

    return _emb(user_id, item_id, hist_flat, price, user_table, item_table,
                hist_table)
